# baseline (speedup 1.0000x reference)
import jax
import jax.numpy as jnp
from jax.experimental import pallas as pl
from jax.experimental.pallas import tpu as pltpu

B, S, D, V = 4, 2048, 2048, 1000
CH = 512
NCH = S // CH
NG = B * NCH


def _head_kernel(x_hbm, w_ref, b_ref, tgt_ref, logits_ref, probs_ref, loss_ref,
                 acc_ref, wbf_ref, xbuf_ref, sems):
    bi = pl.program_id(0)

    @pl.when(bi == 0)
    def _prologue():
        wbf_ref[...] = w_ref[...].astype(jnp.bfloat16)
        acc_ref[0] = 0.0
        acc_ref[1] = 0.0
        pltpu.make_async_copy(x_hbm.at[0, pl.ds(0, CH), :],
                              xbuf_ref.at[0], sems.at[0]).start()
        pltpu.make_async_copy(x_hbm.at[0, pl.ds(CH, CH), :],
                              xbuf_ref.at[1], sems.at[1]).start()

    for k in range(NCH):
        slot = k % 2
        pltpu.make_async_copy(x_hbm.at[bi, pl.ds(k * CH, CH), :],
                              xbuf_ref.at[slot], sems.at[slot]).wait()

        xc = xbuf_ref[slot].astype(jnp.bfloat16)
        lt = jax.lax.dot_general(
            wbf_ref[...], xc, (((1,), (1,)), ((), ())),
            preferred_element_type=jnp.float32)
        lt = lt + b_ref[...]
        logits_ref[0, :, pl.ds(k * CH, CH)] = lt

        m = jnp.max(lt, axis=0, keepdims=True)
        ex = jnp.exp(lt - m)
        s = jnp.sum(ex, axis=0, keepdims=True)
        probs_ref[0, :, pl.ds(k * CH, CH)] = ex * (1.0 / s)

        t = tgt_ref[bi, pl.ds(k * CH, CH)][None, :]
        onehot = (jax.lax.broadcasted_iota(jnp.int32, (V, CH), 0) == t)
        tgt_logit = jnp.sum(jnp.where(onehot, lt, 0.0), axis=0, keepdims=True)
        lse = m + jnp.log(s)
        valid = t >= 0
        nll = jnp.where(valid, lse - tgt_logit, 0.0)
        acc_ref[0] += jnp.sum(nll)
        acc_ref[1] += jnp.sum(valid.astype(jnp.float32))

        nb = bi + (k + 2) // NCH
        nk = (k + 2) % NCH

        @pl.when(nb < B)
        def _prefetch():
            pltpu.make_async_copy(x_hbm.at[nb, pl.ds(nk * CH, CH), :],
                                  xbuf_ref.at[slot], sems.at[slot]).start()

    @pl.when(bi == B - 1)
    def _fin():
        val = acc_ref[0] / jnp.maximum(acc_ref[1], 1.0)
        loss_ref[...] = jnp.broadcast_to(val, (1, 1))


@jax.jit
def _head(x, w, b, tgt):
    logits_t, probs_t, loss = pl.pallas_call(
        _head_kernel,
        grid=(B,),
        in_specs=[
            pl.BlockSpec(memory_space=pltpu.MemorySpace.HBM),
            pl.BlockSpec((V, D), lambda i: (0, 0)),
            pl.BlockSpec((V, 1), lambda i: (0, 0)),
            pl.BlockSpec((B, S), lambda i: (0, 0)),
        ],
        out_specs=[
            pl.BlockSpec((1, V, S), lambda i: (i, 0, 0)),
            pl.BlockSpec((1, V, S), lambda i: (i, 0, 0)),
            pl.BlockSpec((1, 1), lambda i: (0, 0)),
        ],
        out_shape=[
            jax.ShapeDtypeStruct((B, V, S), jnp.float32),
            jax.ShapeDtypeStruct((B, V, S), jnp.float32),
            jax.ShapeDtypeStruct((1, 1), jnp.float32),
        ],
        scratch_shapes=[
            pltpu.SMEM((2,), jnp.float32),
            pltpu.VMEM((V, D), jnp.bfloat16),
            pltpu.VMEM((2, CH, D), jnp.float32),
            pltpu.SemaphoreType.DMA((2,)),
        ],
    )(x, w, b, tgt)
    return (jnp.swapaxes(logits_t, 1, 2), jnp.swapaxes(probs_t, 1, 2),
            loss[0, 0])


def kernel(encoder_out, target, target_mask, W, b):
    tgt = jnp.where(target_mask, target, -1).astype(jnp.int32)
    return _head(encoder_out, W, b.reshape(V, 1), tgt)


# speedup vs baseline: 1.1078x; 1.1078x over previous
import jax
import jax.numpy as jnp
from jax.experimental import pallas as pl
from jax.experimental.pallas import tpu as pltpu

B, S, D, V = 4, 2048, 2048, 1000
TILE_S = 1024
NS = S // TILE_S


def _head_kernel(x_ref, w_ref, b_ref, tgt_ref, msk_ref, logits_ref, probs_ref,
                 loss_ref, acc_ref, wbf_ref):
    bi = pl.program_id(0)
    sj = pl.program_id(1)

    @pl.when((bi == 0) & (sj == 0))
    def _cast_w():
        wbf_ref[...] = w_ref[...].astype(jnp.bfloat16)

    x = x_ref[0].astype(jnp.bfloat16)
    logits_t = jax.lax.dot_general(
        wbf_ref[...], x, (((1,), (1,)), ((), ())),
        preferred_element_type=jnp.float32)
    logits_t = logits_t + jnp.swapaxes(b_ref[...], 0, 1)
    logits_ref[0] = logits_t

    m = jnp.max(logits_t, axis=0, keepdims=True)
    ex = jnp.exp(logits_t - m)
    s = jnp.sum(ex, axis=0, keepdims=True)
    probs_ref[0] = ex * (1.0 / s)

    t = tgt_ref[bi, pl.ds(sj * TILE_S, TILE_S)][None, :]
    onehot = (jax.lax.broadcasted_iota(jnp.int32, (V, TILE_S), 0) == t)
    tgt_logit = jnp.sum(jnp.where(onehot, logits_t, 0.0), axis=0, keepdims=True)
    lse = m + jnp.log(s)
    valid = msk_ref[bi, pl.ds(sj * TILE_S, TILE_S)][None, :]
    nll = jnp.where(valid, lse - tgt_logit, 0.0)

    tile_sum = jnp.sum(nll)
    tile_cnt = jnp.sum(valid.astype(jnp.float32))

    @pl.when((bi == 0) & (sj == 0))
    def _init():
        acc_ref[0] = 0.0
        acc_ref[1] = 0.0

    acc_ref[0] += tile_sum
    acc_ref[1] += tile_cnt

    @pl.when((bi == B - 1) & (sj == NS - 1))
    def _fin():
        val = acc_ref[0] / jnp.maximum(acc_ref[1], 1.0)
        loss_ref[...] = jnp.broadcast_to(val, (1, 1))


@jax.jit
def _head(x, w, b, tgt, msk):
    logits_t, probs_t, loss = pl.pallas_call(
        _head_kernel,
        grid=(B, NS),
        in_specs=[
            pl.BlockSpec((1, TILE_S, D), lambda i, j: (i, j, 0)),
            pl.BlockSpec((V, D), lambda i, j: (0, 0)),
            pl.BlockSpec((1, V), lambda i, j: (0, 0)),
            pl.BlockSpec((B, S), lambda i, j: (0, 0)),
            pl.BlockSpec((B, S), lambda i, j: (0, 0)),
        ],
        out_specs=[
            pl.BlockSpec((1, V, TILE_S), lambda i, j: (i, 0, j)),
            pl.BlockSpec((1, V, TILE_S), lambda i, j: (i, 0, j)),
            pl.BlockSpec((1, 1), lambda i, j: (0, 0)),
        ],
        out_shape=[
            jax.ShapeDtypeStruct((B, V, S), jnp.float32),
            jax.ShapeDtypeStruct((B, V, S), jnp.float32),
            jax.ShapeDtypeStruct((1, 1), jnp.float32),
        ],
        scratch_shapes=[pltpu.SMEM((2,), jnp.float32),
                        pltpu.VMEM((V, D), jnp.bfloat16)],
    )(x, w, b, tgt, msk)
    return (jnp.swapaxes(logits_t, 1, 2), jnp.swapaxes(probs_t, 1, 2),
            loss[0, 0])


def kernel(encoder_out, target, target_mask, W, b):
    return _head(encoder_out, W, b.reshape(1, V), target, target_mask)
